# baseline (speedup 1.0000x reference)
import functools
import math

import jax
import jax.numpy as jnp
from jax import lax
from jax.experimental import pallas as pl
from jax.experimental.pallas import tpu as pltpu
from jax.experimental.pallas import tpu_sc as plsc

D = 128
B = 1024
T = 200
N = B * T
NW = 32
CH = 64
SIZES = (48, 48, 48, 40, 16)
_INV_SQRT_D = 1.0 / math.sqrt(D)


@functools.cache
def _sc_gather(n_rows):
    per_w = n_rows // NW
    nchw = per_w // CH

    def body(t0, t1, t2, t3, i0, i1, i2, i3,
             o0, o1, idx_v, rows_v, pk_v,
             g0, g1, g2, g3, s0, s1):
        tables = (t0, t1, t2, t3)
        outs = (o0, o1)
        gsems = (g0, g1, g2, g3)
        ssems = (s0, s1)
        wid = lax.axis_index("s") * 2 + lax.axis_index("c")
        base = wid * per_w
        for tab, iref in enumerate((i0, i1, i2, i3)):
            pltpu.sync_copy(iref.at[wid], idx_v.at[tab])

        def gathers(c, b):
            return [pltpu.async_copy(tables[tab].at[idx_v.at[tab, c]],
                                     rows_v.at[tab, b], gsems[tab])
                    for tab in range(4)]

        def pack_chunk(b):
            def prow(r, carry):
                for pair in range(2):
                    for g in range(8):
                        a = rows_v[2 * pair, b, r, pl.ds(16 * g, 16)]
                        z = rows_v[2 * pair + 1, b, r, pl.ds(16 * g, 16)]
                        ai = lax.bitcast_convert_type(a, jnp.int32)
                        zi = lax.bitcast_convert_type(z, jnp.int32)
                        ar = lax.shift_right_logical(ai + 0x8000, 16)
                        zr = (zi + 0x8000) & jnp.int32(-65536)
                        pk_v[pair, b, r, pl.ds(16 * g, 16)] = ar | zr
                return carry
            lax.fori_loop(0, CH, prow, 0)

        def issue_stores(c, b):
            row0 = base + c * CH
            for pair in range(2):
                pltpu.async_copy(pk_v.at[pair, b],
                                 outs[pair].at[pl.ds(row0, CH)], ssems[pair])

        def wait_stores(b):
            for pair in range(2):
                pltpu.make_async_copy(pk_v.at[pair, b],
                                      outs[pair].at[pl.ds(0, CH)],
                                      ssems[pair]).wait()

        def step(s, carry):
            c0 = s * 2

            @pl.when(s > 0)
            def _():
                wait_stores(0)
            hg0 = gathers(c0, 0)

            @pl.when(s > 0)
            def _():
                wait_stores(1)
            hg1 = gathers(c0 + 1, 1)
            for h in hg0:
                h.wait()
            pack_chunk(0)
            issue_stores(c0, 0)
            for h in hg1:
                h.wait()
            pack_chunk(1)
            issue_stores(c0 + 1, 1)
            return carry

        lax.fori_loop(0, nchw // 2, step, 0)
        wait_stores(0)
        wait_stores(1)

    return pl.kernel(
        body,
        out_type=tuple(jax.ShapeDtypeStruct((n_rows, D), jnp.int32)
                       for _ in range(2)),
        mesh=plsc.VectorSubcoreMesh(core_axis_name="c", subcore_axis_name="s"),
        scratch_types=[
            pltpu.VMEM((4, nchw, CH), jnp.int32),
            pltpu.VMEM((4, 2, CH, D), jnp.float32),
            pltpu.VMEM((2, 2, CH, D), jnp.int32),
        ] + [pltpu.SemaphoreType.DMA] * 6,
    )


def _ln(x, gm, bt):
    mean = jnp.mean(x, axis=-1, keepdims=True)
    xc = x - mean
    var = jnp.mean(xc * xc, axis=-1, keepdims=True)
    return xc * lax.rsqrt(var + 1e-5) * gm + bt


def _ev_block(frp, g01, g23, dts, ats, w0, w1, w2, w3, m2, b2):
    acc = None
    for g, wl, wh in ((g01, w0, w1), (g23, w2, w3)):
        x = g[...]
        lo = lax.bitcast_convert_type(x << 16, jnp.float32).astype(jnp.bfloat16)
        hi = lax.bitcast_convert_type(x & jnp.int32(-65536),
                                      jnp.float32).astype(jnp.bfloat16)
        p = jnp.dot(lo, wl[...], preferred_element_type=jnp.float32) \
            + jnp.dot(hi, wh[...], preferred_element_type=jnp.float32)
        acc = p if acc is None else acc + p

    tdl = dts[...].reshape(B)
    tal = ats[...].reshape(B)
    feats = [tdl, tal]
    for j in range(7):
        feats.append(jnp.sin(tdl * frp[j] + frp[8 + j]))
    for j in range(7):
        feats.append(jnp.sin(tal * frp[16 + j] + frp[24 + j]))
    s = jnp.stack(feats, axis=0)
    c = lax.dot_general(s, m2[...], (((0,), (0,)), ((), ())),
                        preferred_element_type=jnp.float32)
    return (acc + c + b2[...]) * _INV_SQRT_D


def _tc_body_ctx(frp, g01, g23, dts, ats, pc,
                 w0, w1, w2, w3, m2, b2, cw, ct, gm, bt, out_ref):
    i = pl.program_id(0)

    @pl.when(i == 0)
    def _():
        ctx = jnp.dot(pc[...], cw[...],
                      preferred_element_type=jnp.float32) + ct[...]
        out_ref[...] = _ln(ctx, gm[...], bt[...])

    @pl.when(i > 0)
    def _():
        ev = _ev_block(frp, g01, g23, dts, ats, w0, w1, w2, w3, m2, b2)
        out_ref[...] = _ln(ev, gm[...], bt[...])


def _tc_body_ev(frp, g01, g23, dts, ats,
                w0, w1, w2, w3, m2, b2, gm, bt, prev, out_ref):
    ev = _ev_block(frp, g01, g23, dts, ats, w0, w1, w2, w3, m2, b2)
    out_ref[...] = _ln(ev, gm[...], bt[...])


_FULL = lambda shape: pl.BlockSpec(shape, lambda i: (0,) * len(shape))
_WSPECS = [_FULL((D, D))] * 4 + [_FULL((16, D)), _FULL((1, D))]


def _tc_half1(frp, g01, g23, dts, ats, pc,
              w0, w1, w2, w3, m2, b2, cw, ct, gm, bt, nt):
    gmap = lambda i: (jnp.maximum(i - 1, 0), 0)
    tmap = lambda i: (jnp.maximum(i - 1, 0), 0, 0)
    return pl.pallas_call(
        _tc_body_ctx,
        grid=(nt + 1,),
        in_specs=[
            pl.BlockSpec(memory_space=pltpu.SMEM),
        ] + [pl.BlockSpec((B, D), gmap)] * 2 + [
            pl.BlockSpec((1, 1, B), tmap),
            pl.BlockSpec((1, 1, B), tmap),
            _FULL((B, 64)),
        ] + _WSPECS + [
            _FULL((64, D)), _FULL((1, D)), _FULL((1, D)), _FULL((1, D)),
        ],
        out_specs=pl.BlockSpec((B, D), lambda i: (i, 0)),
        out_shape=jax.ShapeDtypeStruct(((T + 1) * B, D), jnp.float32),
    )(frp, g01, g23, dts, ats, pc,
      w0, w1, w2, w3, m2, b2, cw, ct, gm, bt)


def _tc_half2(frp, g01, g23, dts, ats,
              w0, w1, w2, w3, m2, b2, gm, bt, prev, off, nt):
    gmap = lambda i: (i, 0)
    tmap = lambda i: (i, 0, 0)
    return pl.pallas_call(
        _tc_body_ev,
        grid=(nt,),
        in_specs=[
            pl.BlockSpec(memory_space=pltpu.SMEM),
        ] + [pl.BlockSpec((B, D), gmap)] * 2 + [
            pl.BlockSpec((1, 1, B), tmap),
            pl.BlockSpec((1, 1, B), tmap),
        ] + _WSPECS + [
            _FULL((1, D)), _FULL((1, D)),
            pl.BlockSpec(memory_space=pl.ANY),
        ],
        out_specs=pl.BlockSpec((B, D), lambda i: (i + off, 0)),
        out_shape=jax.ShapeDtypeStruct(((T + 1) * B, D), jnp.float32),
        input_output_aliases={13: 0},
    )(frp, g01, g23, dts, ats,
      w0, w1, w2, w3, m2, b2, gm, bt, prev)


def _prep_params(rel_lin_w, rel_lin_b, rel_freq_w, rel_freq_b,
                 abs_lin_w, abs_lin_b, abs_freq_w, abs_freq_b,
                 time_proj_w, ctx_token, context_proj_w,
                 final_proj_w, final_proj_b, ln_gamma, ln_beta):
    wt = final_proj_w[:, 4 * D:5 * D]
    m = time_proj_w.T @ wt.T
    b2 = (final_proj_b + rel_lin_b[0] * m[0] + abs_lin_b[0] * m[8]).reshape(1, D)
    m2 = jnp.concatenate([
        (rel_lin_w[0, 0] * m[0]).reshape(1, D),
        (abs_lin_w[0, 0] * m[8]).reshape(1, D),
        m[1:8], m[9:16]], axis=0)
    frp = jnp.zeros((32,), jnp.float32)
    frp = frp.at[0:7].set(rel_freq_w[:, 0]).at[8:15].set(rel_freq_b)
    frp = frp.at[16:23].set(abs_freq_w[:, 0]).at[24:31].set(abs_freq_b)
    ws = [final_proj_w[:, k * D:(k + 1) * D].T.astype(jnp.bfloat16)
          for k in range(4)]
    cw = context_proj_w.T
    ct = ctx_token.reshape(1, D)
    gm = ln_gamma.reshape(1, D)
    bt = ln_beta.reshape(1, D)
    return frp, ws, m2, b2, cw, ct, gm, bt


def kernel(raw_concept_ids, concept_ids, value_ids, position_ids, delta_ts,
           abs_ts, patient_contexts, raw_table, con_table, val_table,
           pos_table, rel_lin_w, rel_lin_b, rel_freq_w, rel_freq_b,
           abs_lin_w, abs_lin_b, abs_freq_w, abs_freq_b, time_proj_w,
           ctx_token, context_proj_w, final_proj_w, final_proj_b,
           ln_gamma, ln_beta):
    id_in = (raw_concept_ids, concept_ids, value_ids, position_ids)
    starts = [sum(SIZES[:h]) for h in range(len(SIZES))]
    tabs = (raw_table, con_table, val_table, pos_table)
    gh, dts, ats = [], [], []
    for h, nt in enumerate(SIZES):
        s0 = starts[h]
        ids_h = [a[:, s0:s0 + nt].astype(jnp.int32).T.reshape(
            NW, nt * B // NW // CH, CH) for a in id_in]
        gh.append(_sc_gather(nt * B)(*tabs, *ids_h))
        dts.append(delta_ts[:, s0:s0 + nt].T.reshape(nt, 1, B))
        ats.append(abs_ts[:, s0:s0 + nt].T.reshape(nt, 1, B))
    frp, ws, m2, b2, cw, ct, gm, bt = _prep_params(
        rel_lin_w, rel_lin_b, rel_freq_w, rel_freq_b,
        abs_lin_w, abs_lin_b, abs_freq_w, abs_freq_b,
        time_proj_w, ctx_token, context_proj_w,
        final_proj_w, final_proj_b, ln_gamma, ln_beta)
    out = _tc_half1(frp, gh[0][0], gh[0][1], dts[0], ats[0], patient_contexts,
                    *ws, m2, b2, cw, ct, gm, bt, SIZES[0])
    for h in range(1, len(SIZES)):
        out = _tc_half2(frp, gh[h][0], gh[h][1], dts[h], ats[h],
                        *ws, m2, b2, gm, bt, out, 1 + starts[h], SIZES[h])
    return out.reshape(T + 1, B, D).transpose(1, 0, 2)


# speedup vs baseline: 1.0025x; 1.0025x over previous
import functools
import math

import jax
import jax.numpy as jnp
from jax import lax
from jax.experimental import pallas as pl
from jax.experimental.pallas import tpu as pltpu
from jax.experimental.pallas import tpu_sc as plsc

D = 128
B = 1024
T = 200
N = B * T
NW = 32
CH = 64
SIZES = (44, 44, 44, 44, 24)
_INV_SQRT_D = 1.0 / math.sqrt(D)


@functools.cache
def _sc_gather(n_rows):
    per_w = n_rows // NW
    nchw = per_w // CH

    def body(t0, t1, t2, t3, i0, i1, i2, i3,
             o0, o1, idx_v, rows_v, pk_v,
             g0, g1, g2, g3, s0, s1):
        tables = (t0, t1, t2, t3)
        outs = (o0, o1)
        gsems = (g0, g1, g2, g3)
        ssems = (s0, s1)
        wid = lax.axis_index("s") * 2 + lax.axis_index("c")
        base = wid * per_w
        for tab, iref in enumerate((i0, i1, i2, i3)):
            pltpu.sync_copy(iref.at[wid], idx_v.at[tab])

        def gathers(c, b):
            return [pltpu.async_copy(tables[tab].at[idx_v.at[tab, c]],
                                     rows_v.at[tab, b], gsems[tab])
                    for tab in range(4)]

        def pack_chunk(b):
            def prow(r, carry):
                for pair in range(2):
                    for g in range(8):
                        a = rows_v[2 * pair, b, r, pl.ds(16 * g, 16)]
                        z = rows_v[2 * pair + 1, b, r, pl.ds(16 * g, 16)]
                        ai = lax.bitcast_convert_type(a, jnp.int32)
                        zi = lax.bitcast_convert_type(z, jnp.int32)
                        ar = lax.shift_right_logical(ai + 0x8000, 16)
                        zr = (zi + 0x8000) & jnp.int32(-65536)
                        pk_v[pair, b, r, pl.ds(16 * g, 16)] = ar | zr
                return carry
            lax.fori_loop(0, CH, prow, 0)

        def issue_stores(c, b):
            row0 = base + c * CH
            for pair in range(2):
                pltpu.async_copy(pk_v.at[pair, b],
                                 outs[pair].at[pl.ds(row0, CH)], ssems[pair])

        def wait_stores(b):
            for pair in range(2):
                pltpu.make_async_copy(pk_v.at[pair, b],
                                      outs[pair].at[pl.ds(0, CH)],
                                      ssems[pair]).wait()

        def step(s, carry):
            c0 = s * 2

            @pl.when(s > 0)
            def _():
                wait_stores(0)
            hg0 = gathers(c0, 0)

            @pl.when(s > 0)
            def _():
                wait_stores(1)
            hg1 = gathers(c0 + 1, 1)
            for h in hg0:
                h.wait()
            pack_chunk(0)
            issue_stores(c0, 0)
            for h in hg1:
                h.wait()
            pack_chunk(1)
            issue_stores(c0 + 1, 1)
            return carry

        lax.fori_loop(0, nchw // 2, step, 0)
        wait_stores(0)
        wait_stores(1)

    return pl.kernel(
        body,
        out_type=tuple(jax.ShapeDtypeStruct((n_rows, D), jnp.int32)
                       for _ in range(2)),
        mesh=plsc.VectorSubcoreMesh(core_axis_name="c", subcore_axis_name="s"),
        scratch_types=[
            pltpu.VMEM((4, nchw, CH), jnp.int32),
            pltpu.VMEM((4, 2, CH, D), jnp.float32),
            pltpu.VMEM((2, 2, CH, D), jnp.int32),
        ] + [pltpu.SemaphoreType.DMA] * 6,
    )


def _ln(x, gm, bt):
    mean = jnp.mean(x, axis=-1, keepdims=True)
    xc = x - mean
    var = jnp.mean(xc * xc, axis=-1, keepdims=True)
    return xc * lax.rsqrt(var + 1e-5) * gm + bt


def _ev_block(frp, g01, g23, dts, ats, w0, w1, w2, w3, m2, b2):
    acc = None
    for g, wl, wh in ((g01, w0, w1), (g23, w2, w3)):
        x = g[...]
        lo = lax.bitcast_convert_type(x << 16, jnp.float32).astype(jnp.bfloat16)
        hi = lax.bitcast_convert_type(x & jnp.int32(-65536),
                                      jnp.float32).astype(jnp.bfloat16)
        p = jnp.dot(lo, wl[...], preferred_element_type=jnp.float32) \
            + jnp.dot(hi, wh[...], preferred_element_type=jnp.float32)
        acc = p if acc is None else acc + p

    tdl = dts[...].reshape(B)
    tal = ats[...].reshape(B)
    feats = [tdl, tal]
    for j in range(7):
        feats.append(jnp.sin(tdl * frp[j] + frp[8 + j]))
    for j in range(7):
        feats.append(jnp.sin(tal * frp[16 + j] + frp[24 + j]))
    s = jnp.stack(feats, axis=0)
    c = lax.dot_general(s, m2[...], (((0,), (0,)), ((), ())),
                        preferred_element_type=jnp.float32)
    return (acc + c + b2[...]) * _INV_SQRT_D


def _tc_body_ctx(frp, g01, g23, dts, ats, pc,
                 w0, w1, w2, w3, m2, b2, cw, ct, gm, bt, out_ref):
    i = pl.program_id(0)

    @pl.when(i == 0)
    def _():
        ctx = jnp.dot(pc[...], cw[...],
                      preferred_element_type=jnp.float32) + ct[...]
        out_ref[...] = _ln(ctx, gm[...], bt[...])

    @pl.when(i > 0)
    def _():
        ev = _ev_block(frp, g01, g23, dts, ats, w0, w1, w2, w3, m2, b2)
        out_ref[...] = _ln(ev, gm[...], bt[...])


def _tc_body_ev(frp, g01, g23, dts, ats,
                w0, w1, w2, w3, m2, b2, gm, bt, prev, out_ref):
    ev = _ev_block(frp, g01, g23, dts, ats, w0, w1, w2, w3, m2, b2)
    out_ref[...] = _ln(ev, gm[...], bt[...])


_FULL = lambda shape: pl.BlockSpec(shape, lambda i: (0,) * len(shape))
_WSPECS = [_FULL((D, D))] * 4 + [_FULL((16, D)), _FULL((1, D))]


def _tc_half1(frp, g01, g23, dts, ats, pc,
              w0, w1, w2, w3, m2, b2, cw, ct, gm, bt, nt):
    gmap = lambda i: (jnp.maximum(i - 1, 0), 0)
    tmap = lambda i: (jnp.maximum(i - 1, 0), 0, 0)
    return pl.pallas_call(
        _tc_body_ctx,
        grid=(nt + 1,),
        in_specs=[
            pl.BlockSpec(memory_space=pltpu.SMEM),
        ] + [pl.BlockSpec((B, D), gmap)] * 2 + [
            pl.BlockSpec((1, 1, B), tmap),
            pl.BlockSpec((1, 1, B), tmap),
            _FULL((B, 64)),
        ] + _WSPECS + [
            _FULL((64, D)), _FULL((1, D)), _FULL((1, D)), _FULL((1, D)),
        ],
        out_specs=pl.BlockSpec((B, D), lambda i: (i, 0)),
        out_shape=jax.ShapeDtypeStruct(((T + 1) * B, D), jnp.float32),
    )(frp, g01, g23, dts, ats, pc,
      w0, w1, w2, w3, m2, b2, cw, ct, gm, bt)


def _tc_half2(frp, g01, g23, dts, ats,
              w0, w1, w2, w3, m2, b2, gm, bt, prev, off, nt):
    gmap = lambda i: (i, 0)
    tmap = lambda i: (i, 0, 0)
    return pl.pallas_call(
        _tc_body_ev,
        grid=(nt,),
        in_specs=[
            pl.BlockSpec(memory_space=pltpu.SMEM),
        ] + [pl.BlockSpec((B, D), gmap)] * 2 + [
            pl.BlockSpec((1, 1, B), tmap),
            pl.BlockSpec((1, 1, B), tmap),
        ] + _WSPECS + [
            _FULL((1, D)), _FULL((1, D)),
            pl.BlockSpec(memory_space=pl.ANY),
        ],
        out_specs=pl.BlockSpec((B, D), lambda i: (i + off, 0)),
        out_shape=jax.ShapeDtypeStruct(((T + 1) * B, D), jnp.float32),
        input_output_aliases={13: 0},
    )(frp, g01, g23, dts, ats,
      w0, w1, w2, w3, m2, b2, gm, bt, prev)


def _prep_params(rel_lin_w, rel_lin_b, rel_freq_w, rel_freq_b,
                 abs_lin_w, abs_lin_b, abs_freq_w, abs_freq_b,
                 time_proj_w, ctx_token, context_proj_w,
                 final_proj_w, final_proj_b, ln_gamma, ln_beta):
    wt = final_proj_w[:, 4 * D:5 * D]
    m = time_proj_w.T @ wt.T
    b2 = (final_proj_b + rel_lin_b[0] * m[0] + abs_lin_b[0] * m[8]).reshape(1, D)
    m2 = jnp.concatenate([
        (rel_lin_w[0, 0] * m[0]).reshape(1, D),
        (abs_lin_w[0, 0] * m[8]).reshape(1, D),
        m[1:8], m[9:16]], axis=0)
    frp = jnp.zeros((32,), jnp.float32)
    frp = frp.at[0:7].set(rel_freq_w[:, 0]).at[8:15].set(rel_freq_b)
    frp = frp.at[16:23].set(abs_freq_w[:, 0]).at[24:31].set(abs_freq_b)
    ws = [final_proj_w[:, k * D:(k + 1) * D].T.astype(jnp.bfloat16)
          for k in range(4)]
    cw = context_proj_w.T
    ct = ctx_token.reshape(1, D)
    gm = ln_gamma.reshape(1, D)
    bt = ln_beta.reshape(1, D)
    return frp, ws, m2, b2, cw, ct, gm, bt


def kernel(raw_concept_ids, concept_ids, value_ids, position_ids, delta_ts,
           abs_ts, patient_contexts, raw_table, con_table, val_table,
           pos_table, rel_lin_w, rel_lin_b, rel_freq_w, rel_freq_b,
           abs_lin_w, abs_lin_b, abs_freq_w, abs_freq_b, time_proj_w,
           ctx_token, context_proj_w, final_proj_w, final_proj_b,
           ln_gamma, ln_beta):
    id_in = (raw_concept_ids, concept_ids, value_ids, position_ids)
    starts = [sum(SIZES[:h]) for h in range(len(SIZES))]
    tabs = (raw_table, con_table, val_table, pos_table)
    gh, dts, ats = [], [], []
    for h, nt in enumerate(SIZES):
        s0 = starts[h]
        ids_h = [a[:, s0:s0 + nt].astype(jnp.int32).T.reshape(
            NW, nt * B // NW // CH, CH) for a in id_in]
        gh.append(_sc_gather(nt * B)(*tabs, *ids_h))
        dts.append(delta_ts[:, s0:s0 + nt].T.reshape(nt, 1, B))
        ats.append(abs_ts[:, s0:s0 + nt].T.reshape(nt, 1, B))
    frp, ws, m2, b2, cw, ct, gm, bt = _prep_params(
        rel_lin_w, rel_lin_b, rel_freq_w, rel_freq_b,
        abs_lin_w, abs_lin_b, abs_freq_w, abs_freq_b,
        time_proj_w, ctx_token, context_proj_w,
        final_proj_w, final_proj_b, ln_gamma, ln_beta)
    out = _tc_half1(frp, gh[0][0], gh[0][1], dts[0], ats[0], patient_contexts,
                    *ws, m2, b2, cw, ct, gm, bt, SIZES[0])
    for h in range(1, len(SIZES)):
        out = _tc_half2(frp, gh[h][0], gh[h][1], dts[h], ats[h],
                        *ws, m2, b2, gm, bt, out, 1 + starts[h], SIZES[h])
    return out.reshape(T + 1, B, D).transpose(1, 0, 2)


# speedup vs baseline: 1.0068x; 1.0043x over previous
import functools
import math

import jax
import jax.numpy as jnp
from jax import lax
from jax.experimental import pallas as pl
from jax.experimental.pallas import tpu as pltpu
from jax.experimental.pallas import tpu_sc as plsc

D = 128
B = 1024
T = 200
N = B * T
NW = 32
CH = 64
SIZES = (40, 40, 40, 40, 40)
_INV_SQRT_D = 1.0 / math.sqrt(D)


@functools.cache
def _sc_gather(n_rows):
    per_w = n_rows // NW
    nchw = per_w // CH

    def body(t0, t1, t2, t3, i0, i1, i2, i3,
             o0, o1, idx_v, rows_v, pk_v,
             g0, g1, g2, g3, s0, s1):
        tables = (t0, t1, t2, t3)
        outs = (o0, o1)
        gsems = (g0, g1, g2, g3)
        ssems = (s0, s1)
        wid = lax.axis_index("s") * 2 + lax.axis_index("c")
        base = wid * per_w
        for tab, iref in enumerate((i0, i1, i2, i3)):
            pltpu.sync_copy(iref.at[wid], idx_v.at[tab])

        def gathers(c, b):
            return [pltpu.async_copy(tables[tab].at[idx_v.at[tab, c]],
                                     rows_v.at[tab, b], gsems[tab])
                    for tab in range(4)]

        def pack_chunk(b):
            def prow(r, carry):
                for pair in range(2):
                    for g in range(8):
                        a = rows_v[2 * pair, b, r, pl.ds(16 * g, 16)]
                        z = rows_v[2 * pair + 1, b, r, pl.ds(16 * g, 16)]
                        ai = lax.bitcast_convert_type(a, jnp.int32)
                        zi = lax.bitcast_convert_type(z, jnp.int32)
                        ar = lax.shift_right_logical(ai + 0x8000, 16)
                        zr = (zi + 0x8000) & jnp.int32(-65536)
                        pk_v[pair, b, r, pl.ds(16 * g, 16)] = ar | zr
                return carry
            lax.fori_loop(0, CH, prow, 0)

        def issue_stores(c, b):
            row0 = base + c * CH
            for pair in range(2):
                pltpu.async_copy(pk_v.at[pair, b],
                                 outs[pair].at[pl.ds(row0, CH)], ssems[pair])

        def wait_stores(b):
            for pair in range(2):
                pltpu.make_async_copy(pk_v.at[pair, b],
                                      outs[pair].at[pl.ds(0, CH)],
                                      ssems[pair]).wait()

        def step(s, carry):
            c0 = s * 2

            @pl.when(s > 0)
            def _():
                wait_stores(0)
            hg0 = gathers(c0, 0)

            @pl.when(s > 0)
            def _():
                wait_stores(1)
            hg1 = gathers(c0 + 1, 1)
            for h in hg0:
                h.wait()
            pack_chunk(0)
            issue_stores(c0, 0)
            for h in hg1:
                h.wait()
            pack_chunk(1)
            issue_stores(c0 + 1, 1)
            return carry

        lax.fori_loop(0, nchw // 2, step, 0)
        wait_stores(0)
        wait_stores(1)

    return pl.kernel(
        body,
        out_type=tuple(jax.ShapeDtypeStruct((n_rows, D), jnp.int32)
                       for _ in range(2)),
        mesh=plsc.VectorSubcoreMesh(core_axis_name="c", subcore_axis_name="s"),
        scratch_types=[
            pltpu.VMEM((4, nchw, CH), jnp.int32),
            pltpu.VMEM((4, 2, CH, D), jnp.float32),
            pltpu.VMEM((2, 2, CH, D), jnp.int32),
        ] + [pltpu.SemaphoreType.DMA] * 6,
    )


def _ln(x, gm, bt):
    mean = jnp.mean(x, axis=-1, keepdims=True)
    xc = x - mean
    var = jnp.mean(xc * xc, axis=-1, keepdims=True)
    return xc * lax.rsqrt(var + 1e-5) * gm + bt


def _ev_block(frp, g01, g23, dts, ats, w0, w1, w2, w3, m2, b2):
    acc = None
    for g, wl, wh in ((g01, w0, w1), (g23, w2, w3)):
        x = g[...]
        lo = lax.bitcast_convert_type(x << 16, jnp.float32).astype(jnp.bfloat16)
        hi = lax.bitcast_convert_type(x & jnp.int32(-65536),
                                      jnp.float32).astype(jnp.bfloat16)
        p = jnp.dot(lo, wl[...], preferred_element_type=jnp.float32) \
            + jnp.dot(hi, wh[...], preferred_element_type=jnp.float32)
        acc = p if acc is None else acc + p

    tdl = dts[...].reshape(B)
    tal = ats[...].reshape(B)
    feats = [tdl, tal]
    for j in range(7):
        feats.append(jnp.sin(tdl * frp[j] + frp[8 + j]))
    for j in range(7):
        feats.append(jnp.sin(tal * frp[16 + j] + frp[24 + j]))
    s = jnp.stack(feats, axis=0)
    c = lax.dot_general(s, m2[...], (((0,), (0,)), ((), ())),
                        preferred_element_type=jnp.float32)
    return (acc + c + b2[...]) * _INV_SQRT_D


def _tc_body_ctx(frp, g01, g23, dts, ats, pc,
                 w0, w1, w2, w3, m2, b2, cw, ct, gm, bt, out_ref):
    i = pl.program_id(0)

    @pl.when(i == 0)
    def _():
        ctx = jnp.dot(pc[...], cw[...],
                      preferred_element_type=jnp.float32) + ct[...]
        out_ref[...] = _ln(ctx, gm[...], bt[...])

    @pl.when(i > 0)
    def _():
        ev = _ev_block(frp, g01, g23, dts, ats, w0, w1, w2, w3, m2, b2)
        out_ref[...] = _ln(ev, gm[...], bt[...])


def _tc_body_ev(frp, g01, g23, dts, ats,
                w0, w1, w2, w3, m2, b2, gm, bt, prev, out_ref):
    ev = _ev_block(frp, g01, g23, dts, ats, w0, w1, w2, w3, m2, b2)
    out_ref[...] = _ln(ev, gm[...], bt[...])


_FULL = lambda shape: pl.BlockSpec(shape, lambda i: (0,) * len(shape))
_WSPECS = [_FULL((D, D))] * 4 + [_FULL((16, D)), _FULL((1, D))]


def _tc_half1(frp, g01, g23, dts, ats, pc,
              w0, w1, w2, w3, m2, b2, cw, ct, gm, bt, nt):
    gmap = lambda i: (jnp.maximum(i - 1, 0), 0)
    tmap = lambda i: (jnp.maximum(i - 1, 0), 0, 0)
    return pl.pallas_call(
        _tc_body_ctx,
        grid=(nt + 1,),
        in_specs=[
            pl.BlockSpec(memory_space=pltpu.SMEM),
        ] + [pl.BlockSpec((B, D), gmap)] * 2 + [
            pl.BlockSpec((1, 1, B), tmap),
            pl.BlockSpec((1, 1, B), tmap),
            _FULL((B, 64)),
        ] + _WSPECS + [
            _FULL((64, D)), _FULL((1, D)), _FULL((1, D)), _FULL((1, D)),
        ],
        out_specs=pl.BlockSpec((B, D), lambda i: (i, 0)),
        out_shape=jax.ShapeDtypeStruct(((T + 1) * B, D), jnp.float32),
    )(frp, g01, g23, dts, ats, pc,
      w0, w1, w2, w3, m2, b2, cw, ct, gm, bt)


def _tc_half2(frp, g01, g23, dts, ats,
              w0, w1, w2, w3, m2, b2, gm, bt, prev, off, nt):
    gmap = lambda i: (i, 0)
    tmap = lambda i: (i, 0, 0)
    return pl.pallas_call(
        _tc_body_ev,
        grid=(nt,),
        in_specs=[
            pl.BlockSpec(memory_space=pltpu.SMEM),
        ] + [pl.BlockSpec((B, D), gmap)] * 2 + [
            pl.BlockSpec((1, 1, B), tmap),
            pl.BlockSpec((1, 1, B), tmap),
        ] + _WSPECS + [
            _FULL((1, D)), _FULL((1, D)),
            pl.BlockSpec(memory_space=pl.ANY),
        ],
        out_specs=pl.BlockSpec((B, D), lambda i: (i + off, 0)),
        out_shape=jax.ShapeDtypeStruct(((T + 1) * B, D), jnp.float32),
        input_output_aliases={13: 0},
    )(frp, g01, g23, dts, ats,
      w0, w1, w2, w3, m2, b2, gm, bt, prev)


def _prep_params(rel_lin_w, rel_lin_b, rel_freq_w, rel_freq_b,
                 abs_lin_w, abs_lin_b, abs_freq_w, abs_freq_b,
                 time_proj_w, ctx_token, context_proj_w,
                 final_proj_w, final_proj_b, ln_gamma, ln_beta):
    wt = final_proj_w[:, 4 * D:5 * D]
    m = time_proj_w.T @ wt.T
    b2 = (final_proj_b + rel_lin_b[0] * m[0] + abs_lin_b[0] * m[8]).reshape(1, D)
    m2 = jnp.concatenate([
        (rel_lin_w[0, 0] * m[0]).reshape(1, D),
        (abs_lin_w[0, 0] * m[8]).reshape(1, D),
        m[1:8], m[9:16]], axis=0)
    frp = jnp.zeros((32,), jnp.float32)
    frp = frp.at[0:7].set(rel_freq_w[:, 0]).at[8:15].set(rel_freq_b)
    frp = frp.at[16:23].set(abs_freq_w[:, 0]).at[24:31].set(abs_freq_b)
    ws = [final_proj_w[:, k * D:(k + 1) * D].T.astype(jnp.bfloat16)
          for k in range(4)]
    cw = context_proj_w.T
    ct = ctx_token.reshape(1, D)
    gm = ln_gamma.reshape(1, D)
    bt = ln_beta.reshape(1, D)
    return frp, ws, m2, b2, cw, ct, gm, bt


def kernel(raw_concept_ids, concept_ids, value_ids, position_ids, delta_ts,
           abs_ts, patient_contexts, raw_table, con_table, val_table,
           pos_table, rel_lin_w, rel_lin_b, rel_freq_w, rel_freq_b,
           abs_lin_w, abs_lin_b, abs_freq_w, abs_freq_b, time_proj_w,
           ctx_token, context_proj_w, final_proj_w, final_proj_b,
           ln_gamma, ln_beta):
    id_in = (raw_concept_ids, concept_ids, value_ids, position_ids)
    starts = [sum(SIZES[:h]) for h in range(len(SIZES))]
    tabs = (raw_table, con_table, val_table, pos_table)
    gh, dts, ats = [], [], []
    for h, nt in enumerate(SIZES):
        s0 = starts[h]
        ids_h = [a[:, s0:s0 + nt].astype(jnp.int32).T.reshape(
            NW, nt * B // NW // CH, CH) for a in id_in]
        gh.append(_sc_gather(nt * B)(*tabs, *ids_h))
        dts.append(delta_ts[:, s0:s0 + nt].T.reshape(nt, 1, B))
        ats.append(abs_ts[:, s0:s0 + nt].T.reshape(nt, 1, B))
    frp, ws, m2, b2, cw, ct, gm, bt = _prep_params(
        rel_lin_w, rel_lin_b, rel_freq_w, rel_freq_b,
        abs_lin_w, abs_lin_b, abs_freq_w, abs_freq_b,
        time_proj_w, ctx_token, context_proj_w,
        final_proj_w, final_proj_b, ln_gamma, ln_beta)
    out = _tc_half1(frp, gh[0][0], gh[0][1], dts[0], ats[0], patient_contexts,
                    *ws, m2, b2, cw, ct, gm, bt, SIZES[0])
    for h in range(1, len(SIZES)):
        out = _tc_half2(frp, gh[h][0], gh[h][1], dts[h], ats[h],
                        *ws, m2, b2, gm, bt, out, 1 + starts[h], SIZES[h])
    return out.reshape(T + 1, B, D).transpose(1, 0, 2)


# speedup vs baseline: 1.0144x; 1.0075x over previous
import functools
import math

import jax
import jax.numpy as jnp
from jax import lax
from jax.experimental import pallas as pl
from jax.experimental.pallas import tpu as pltpu
from jax.experimental.pallas import tpu_sc as plsc

D = 128
B = 1024
T = 200
N = B * T
NW = 32
CH = 64
SIZES = (40, 40, 40, 40, 40)
_INV_SQRT_D = 1.0 / math.sqrt(D)


@functools.cache
def _sc_gather(n_rows):
    per_w = n_rows // NW
    nchw = per_w // CH

    def body(t0, t1, t2, t3, i0, i1, i2, i3,
             o0, o1, idx_v, rows_v, pk_v,
             g0, g1, g2, g3, s0, s1):
        tables = (t0, t1, t2, t3)
        outs = (o0, o1)
        gsems = (g0, g1, g2, g3)
        ssems = (s0, s1)
        wid = lax.axis_index("s") * 2 + lax.axis_index("c")
        base = wid * per_w
        ih = [pltpu.async_copy(iref.at[wid], idx_v.at[tab], gsems[tab])
              for tab, iref in enumerate((i0, i1, i2, i3))]
        for h in ih:
            h.wait()

        def gathers(c, b):
            return [pltpu.async_copy(tables[tab].at[idx_v.at[tab, c]],
                                     rows_v.at[tab, b], gsems[tab])
                    for tab in range(4)]

        def pack_chunk(b):
            def prow(r, carry):
                for pair in range(2):
                    for g in range(8):
                        a = rows_v[2 * pair, b, r, pl.ds(16 * g, 16)]
                        z = rows_v[2 * pair + 1, b, r, pl.ds(16 * g, 16)]
                        ai = lax.bitcast_convert_type(a, jnp.int32)
                        zi = lax.bitcast_convert_type(z, jnp.int32)
                        ar = lax.shift_right_logical(ai + 0x8000, 16)
                        zr = (zi + 0x8000) & jnp.int32(-65536)
                        pk_v[pair, b, r, pl.ds(16 * g, 16)] = ar | zr
                return carry
            lax.fori_loop(0, CH, prow, 0)

        def issue_stores(c, b):
            row0 = base + c * CH
            for pair in range(2):
                pltpu.async_copy(pk_v.at[pair, b],
                                 outs[pair].at[pl.ds(row0, CH)], ssems[pair])

        def wait_stores(b):
            for pair in range(2):
                pltpu.make_async_copy(pk_v.at[pair, b],
                                      outs[pair].at[pl.ds(0, CH)],
                                      ssems[pair]).wait()

        def step(s, carry):
            c0 = s * 2

            @pl.when(s > 0)
            def _():
                wait_stores(0)
            hg0 = gathers(c0, 0)

            @pl.when(s > 0)
            def _():
                wait_stores(1)
            hg1 = gathers(c0 + 1, 1)
            for h in hg0:
                h.wait()
            pack_chunk(0)
            issue_stores(c0, 0)
            for h in hg1:
                h.wait()
            pack_chunk(1)
            issue_stores(c0 + 1, 1)
            return carry

        lax.fori_loop(0, nchw // 2, step, 0)
        wait_stores(0)
        wait_stores(1)

    return pl.kernel(
        body,
        out_type=tuple(jax.ShapeDtypeStruct((n_rows, D), jnp.int32)
                       for _ in range(2)),
        mesh=plsc.VectorSubcoreMesh(core_axis_name="c", subcore_axis_name="s"),
        scratch_types=[
            pltpu.VMEM((4, nchw, CH), jnp.int32),
            pltpu.VMEM((4, 2, CH, D), jnp.float32),
            pltpu.VMEM((2, 2, CH, D), jnp.int32),
        ] + [pltpu.SemaphoreType.DMA] * 6,
    )


def _ln(x, gm, bt):
    mean = jnp.mean(x, axis=-1, keepdims=True)
    xc = x - mean
    var = jnp.mean(xc * xc, axis=-1, keepdims=True)
    return xc * lax.rsqrt(var + 1e-5) * gm + bt


def _ev_block(frp, g01, g23, dts, ats, w0, w1, w2, w3, m2, b2):
    acc = None
    for g, wl, wh in ((g01, w0, w1), (g23, w2, w3)):
        x = g[...]
        lo = lax.bitcast_convert_type(x << 16, jnp.float32).astype(jnp.bfloat16)
        hi = lax.bitcast_convert_type(x & jnp.int32(-65536),
                                      jnp.float32).astype(jnp.bfloat16)
        p = jnp.dot(lo, wl[...], preferred_element_type=jnp.float32) \
            + jnp.dot(hi, wh[...], preferred_element_type=jnp.float32)
        acc = p if acc is None else acc + p

    tdl = dts[...].reshape(B)
    tal = ats[...].reshape(B)
    feats = [tdl, tal]
    for j in range(7):
        feats.append(jnp.sin(tdl * frp[j] + frp[8 + j]))
    for j in range(7):
        feats.append(jnp.sin(tal * frp[16 + j] + frp[24 + j]))
    s = jnp.stack(feats, axis=0)
    c = lax.dot_general(s, m2[...], (((0,), (0,)), ((), ())),
                        preferred_element_type=jnp.float32)
    return (acc + c + b2[...]) * _INV_SQRT_D


def _tc_body_ctx(frp, g01, g23, dts, ats, pc,
                 w0, w1, w2, w3, m2, b2, cw, ct, gm, bt, out_ref):
    i = pl.program_id(0)

    @pl.when(i == 0)
    def _():
        ctx = jnp.dot(pc[...], cw[...],
                      preferred_element_type=jnp.float32) + ct[...]
        out_ref[...] = _ln(ctx, gm[...], bt[...])

    @pl.when(i > 0)
    def _():
        ev = _ev_block(frp, g01, g23, dts, ats, w0, w1, w2, w3, m2, b2)
        out_ref[...] = _ln(ev, gm[...], bt[...])


def _tc_body_ev(frp, g01, g23, dts, ats,
                w0, w1, w2, w3, m2, b2, gm, bt, prev, out_ref):
    ev = _ev_block(frp, g01, g23, dts, ats, w0, w1, w2, w3, m2, b2)
    out_ref[...] = _ln(ev, gm[...], bt[...])


_FULL = lambda shape: pl.BlockSpec(shape, lambda i: (0,) * len(shape))
_WSPECS = [_FULL((D, D))] * 4 + [_FULL((16, D)), _FULL((1, D))]


def _tc_half1(frp, g01, g23, dts, ats, pc,
              w0, w1, w2, w3, m2, b2, cw, ct, gm, bt, nt):
    gmap = lambda i: (jnp.maximum(i - 1, 0), 0)
    tmap = lambda i: (jnp.maximum(i - 1, 0), 0, 0)
    return pl.pallas_call(
        _tc_body_ctx,
        grid=(nt + 1,),
        in_specs=[
            pl.BlockSpec(memory_space=pltpu.SMEM),
        ] + [pl.BlockSpec((B, D), gmap)] * 2 + [
            pl.BlockSpec((1, 1, B), tmap),
            pl.BlockSpec((1, 1, B), tmap),
            _FULL((B, 64)),
        ] + _WSPECS + [
            _FULL((64, D)), _FULL((1, D)), _FULL((1, D)), _FULL((1, D)),
        ],
        out_specs=pl.BlockSpec((B, D), lambda i: (i, 0)),
        out_shape=jax.ShapeDtypeStruct(((T + 1) * B, D), jnp.float32),
    )(frp, g01, g23, dts, ats, pc,
      w0, w1, w2, w3, m2, b2, cw, ct, gm, bt)


def _tc_half2(frp, g01, g23, dts, ats,
              w0, w1, w2, w3, m2, b2, gm, bt, prev, off, nt):
    gmap = lambda i: (i, 0)
    tmap = lambda i: (i, 0, 0)
    return pl.pallas_call(
        _tc_body_ev,
        grid=(nt,),
        in_specs=[
            pl.BlockSpec(memory_space=pltpu.SMEM),
        ] + [pl.BlockSpec((B, D), gmap)] * 2 + [
            pl.BlockSpec((1, 1, B), tmap),
            pl.BlockSpec((1, 1, B), tmap),
        ] + _WSPECS + [
            _FULL((1, D)), _FULL((1, D)),
            pl.BlockSpec(memory_space=pl.ANY),
        ],
        out_specs=pl.BlockSpec((B, D), lambda i: (i + off, 0)),
        out_shape=jax.ShapeDtypeStruct(((T + 1) * B, D), jnp.float32),
        input_output_aliases={13: 0},
    )(frp, g01, g23, dts, ats,
      w0, w1, w2, w3, m2, b2, gm, bt, prev)


def _prep_params(rel_lin_w, rel_lin_b, rel_freq_w, rel_freq_b,
                 abs_lin_w, abs_lin_b, abs_freq_w, abs_freq_b,
                 time_proj_w, ctx_token, context_proj_w,
                 final_proj_w, final_proj_b, ln_gamma, ln_beta):
    wt = final_proj_w[:, 4 * D:5 * D]
    m = time_proj_w.T @ wt.T
    b2 = (final_proj_b + rel_lin_b[0] * m[0] + abs_lin_b[0] * m[8]).reshape(1, D)
    m2 = jnp.concatenate([
        (rel_lin_w[0, 0] * m[0]).reshape(1, D),
        (abs_lin_w[0, 0] * m[8]).reshape(1, D),
        m[1:8], m[9:16]], axis=0)
    frp = jnp.zeros((32,), jnp.float32)
    frp = frp.at[0:7].set(rel_freq_w[:, 0]).at[8:15].set(rel_freq_b)
    frp = frp.at[16:23].set(abs_freq_w[:, 0]).at[24:31].set(abs_freq_b)
    ws = [final_proj_w[:, k * D:(k + 1) * D].T.astype(jnp.bfloat16)
          for k in range(4)]
    cw = context_proj_w.T
    ct = ctx_token.reshape(1, D)
    gm = ln_gamma.reshape(1, D)
    bt = ln_beta.reshape(1, D)
    return frp, ws, m2, b2, cw, ct, gm, bt


def kernel(raw_concept_ids, concept_ids, value_ids, position_ids, delta_ts,
           abs_ts, patient_contexts, raw_table, con_table, val_table,
           pos_table, rel_lin_w, rel_lin_b, rel_freq_w, rel_freq_b,
           abs_lin_w, abs_lin_b, abs_freq_w, abs_freq_b, time_proj_w,
           ctx_token, context_proj_w, final_proj_w, final_proj_b,
           ln_gamma, ln_beta):
    id_in = (raw_concept_ids, concept_ids, value_ids, position_ids)
    starts = [sum(SIZES[:h]) for h in range(len(SIZES))]
    tabs = (raw_table, con_table, val_table, pos_table)
    gh, dts, ats = [], [], []
    for h, nt in enumerate(SIZES):
        s0 = starts[h]
        ids_h = [a[:, s0:s0 + nt].astype(jnp.int32).T.reshape(
            NW, nt * B // NW // CH, CH) for a in id_in]
        gh.append(_sc_gather(nt * B)(*tabs, *ids_h))
        dts.append(delta_ts[:, s0:s0 + nt].T.reshape(nt, 1, B))
        ats.append(abs_ts[:, s0:s0 + nt].T.reshape(nt, 1, B))
    frp, ws, m2, b2, cw, ct, gm, bt = _prep_params(
        rel_lin_w, rel_lin_b, rel_freq_w, rel_freq_b,
        abs_lin_w, abs_lin_b, abs_freq_w, abs_freq_b,
        time_proj_w, ctx_token, context_proj_w,
        final_proj_w, final_proj_b, ln_gamma, ln_beta)
    out = _tc_half1(frp, gh[0][0], gh[0][1], dts[0], ats[0], patient_contexts,
                    *ws, m2, b2, cw, ct, gm, bt, SIZES[0])
    for h in range(1, len(SIZES)):
        out = _tc_half2(frp, gh[h][0], gh[h][1], dts[h], ats[h],
                        *ws, m2, b2, gm, bt, out, 1 + starts[h], SIZES[h])
    return out.reshape(T + 1, B, D).transpose(1, 0, 2)


# speedup vs baseline: 1.0175x; 1.0031x over previous
import functools
import math

import jax
import jax.numpy as jnp
from jax import lax
from jax.experimental import pallas as pl
from jax.experimental.pallas import tpu as pltpu
from jax.experimental.pallas import tpu_sc as plsc

D = 128
B = 1024
T = 200
N = B * T
NW = 32
CH = 64
SIZES = (36, 36, 36, 32, 32, 28)
_INV_SQRT_D = 1.0 / math.sqrt(D)


@functools.cache
def _sc_gather(n_rows):
    per_w = n_rows // NW
    nchw = per_w // CH

    def body(t0, t1, t2, t3, i0, i1, i2, i3,
             o0, o1, idx_v, rows_v, pk_v,
             g0, g1, g2, g3, s0, s1):
        tables = (t0, t1, t2, t3)
        outs = (o0, o1)
        gsems = (g0, g1, g2, g3)
        ssems = (s0, s1)
        wid = lax.axis_index("s") * 2 + lax.axis_index("c")
        base = wid * per_w
        ih = [pltpu.async_copy(iref.at[wid], idx_v.at[tab], gsems[tab])
              for tab, iref in enumerate((i0, i1, i2, i3))]
        for h in ih:
            h.wait()

        def gathers(c, b):
            return [pltpu.async_copy(tables[tab].at[idx_v.at[tab, c]],
                                     rows_v.at[tab, b], gsems[tab])
                    for tab in range(4)]

        def pack_chunk(b):
            def prow(r, carry):
                for pair in range(2):
                    for g in range(8):
                        a = rows_v[2 * pair, b, r, pl.ds(16 * g, 16)]
                        z = rows_v[2 * pair + 1, b, r, pl.ds(16 * g, 16)]
                        ai = lax.bitcast_convert_type(a, jnp.int32)
                        zi = lax.bitcast_convert_type(z, jnp.int32)
                        ar = lax.shift_right_logical(ai + 0x8000, 16)
                        zr = (zi + 0x8000) & jnp.int32(-65536)
                        pk_v[pair, b, r, pl.ds(16 * g, 16)] = ar | zr
                return carry
            lax.fori_loop(0, CH, prow, 0)

        def issue_stores(c, b):
            row0 = base + c * CH
            for pair in range(2):
                pltpu.async_copy(pk_v.at[pair, b],
                                 outs[pair].at[pl.ds(row0, CH)], ssems[pair])

        def wait_stores(b):
            for pair in range(2):
                pltpu.make_async_copy(pk_v.at[pair, b],
                                      outs[pair].at[pl.ds(0, CH)],
                                      ssems[pair]).wait()

        def step(s, carry):
            c0 = s * 2

            @pl.when(s > 0)
            def _():
                wait_stores(0)
            hg0 = gathers(c0, 0)

            @pl.when(s > 0)
            def _():
                wait_stores(1)
            hg1 = gathers(c0 + 1, 1)
            for h in hg0:
                h.wait()
            pack_chunk(0)
            issue_stores(c0, 0)
            for h in hg1:
                h.wait()
            pack_chunk(1)
            issue_stores(c0 + 1, 1)
            return carry

        lax.fori_loop(0, nchw // 2, step, 0)
        wait_stores(0)
        wait_stores(1)

    return pl.kernel(
        body,
        out_type=tuple(jax.ShapeDtypeStruct((n_rows, D), jnp.int32)
                       for _ in range(2)),
        mesh=plsc.VectorSubcoreMesh(core_axis_name="c", subcore_axis_name="s"),
        scratch_types=[
            pltpu.VMEM((4, nchw, CH), jnp.int32),
            pltpu.VMEM((4, 2, CH, D), jnp.float32),
            pltpu.VMEM((2, 2, CH, D), jnp.int32),
        ] + [pltpu.SemaphoreType.DMA] * 6,
    )


def _ln(x, gm, bt):
    mean = jnp.mean(x, axis=-1, keepdims=True)
    xc = x - mean
    var = jnp.mean(xc * xc, axis=-1, keepdims=True)
    return xc * lax.rsqrt(var + 1e-5) * gm + bt


def _ev_block(frp, g01, g23, dts, ats, w0, w1, w2, w3, m2, b2):
    acc = None
    for g, wl, wh in ((g01, w0, w1), (g23, w2, w3)):
        x = g[...]
        lo = lax.bitcast_convert_type(x << 16, jnp.float32).astype(jnp.bfloat16)
        hi = lax.bitcast_convert_type(x & jnp.int32(-65536),
                                      jnp.float32).astype(jnp.bfloat16)
        p = jnp.dot(lo, wl[...], preferred_element_type=jnp.float32) \
            + jnp.dot(hi, wh[...], preferred_element_type=jnp.float32)
        acc = p if acc is None else acc + p

    tdl = dts[...].reshape(B)
    tal = ats[...].reshape(B)
    feats = [tdl, tal]
    for j in range(7):
        feats.append(jnp.sin(tdl * frp[j] + frp[8 + j]))
    for j in range(7):
        feats.append(jnp.sin(tal * frp[16 + j] + frp[24 + j]))
    s = jnp.stack(feats, axis=0)
    c = lax.dot_general(s, m2[...], (((0,), (0,)), ((), ())),
                        preferred_element_type=jnp.float32)
    return (acc + c + b2[...]) * _INV_SQRT_D


def _tc_body_ctx(frp, g01, g23, dts, ats, pc,
                 w0, w1, w2, w3, m2, b2, cw, ct, gm, bt, out_ref):
    i = pl.program_id(0)

    @pl.when(i == 0)
    def _():
        ctx = jnp.dot(pc[...], cw[...],
                      preferred_element_type=jnp.float32) + ct[...]
        out_ref[...] = _ln(ctx, gm[...], bt[...])

    @pl.when(i > 0)
    def _():
        ev = _ev_block(frp, g01, g23, dts, ats, w0, w1, w2, w3, m2, b2)
        out_ref[...] = _ln(ev, gm[...], bt[...])


def _tc_body_ev(frp, g01, g23, dts, ats,
                w0, w1, w2, w3, m2, b2, gm, bt, prev, out_ref):
    ev = _ev_block(frp, g01, g23, dts, ats, w0, w1, w2, w3, m2, b2)
    out_ref[...] = _ln(ev, gm[...], bt[...])


_FULL = lambda shape: pl.BlockSpec(shape, lambda i: (0,) * len(shape))
_WSPECS = [_FULL((D, D))] * 4 + [_FULL((16, D)), _FULL((1, D))]


def _tc_half1(frp, g01, g23, dts, ats, pc,
              w0, w1, w2, w3, m2, b2, cw, ct, gm, bt, nt):
    gmap = lambda i: (jnp.maximum(i - 1, 0), 0)
    tmap = lambda i: (jnp.maximum(i - 1, 0), 0, 0)
    return pl.pallas_call(
        _tc_body_ctx,
        grid=(nt + 1,),
        in_specs=[
            pl.BlockSpec(memory_space=pltpu.SMEM),
        ] + [pl.BlockSpec((B, D), gmap)] * 2 + [
            pl.BlockSpec((1, 1, B), tmap),
            pl.BlockSpec((1, 1, B), tmap),
            _FULL((B, 64)),
        ] + _WSPECS + [
            _FULL((64, D)), _FULL((1, D)), _FULL((1, D)), _FULL((1, D)),
        ],
        out_specs=pl.BlockSpec((B, D), lambda i: (i, 0)),
        out_shape=jax.ShapeDtypeStruct(((T + 1) * B, D), jnp.float32),
    )(frp, g01, g23, dts, ats, pc,
      w0, w1, w2, w3, m2, b2, cw, ct, gm, bt)


def _tc_half2(frp, g01, g23, dts, ats,
              w0, w1, w2, w3, m2, b2, gm, bt, prev, off, nt):
    gmap = lambda i: (i, 0)
    tmap = lambda i: (i, 0, 0)
    return pl.pallas_call(
        _tc_body_ev,
        grid=(nt,),
        in_specs=[
            pl.BlockSpec(memory_space=pltpu.SMEM),
        ] + [pl.BlockSpec((B, D), gmap)] * 2 + [
            pl.BlockSpec((1, 1, B), tmap),
            pl.BlockSpec((1, 1, B), tmap),
        ] + _WSPECS + [
            _FULL((1, D)), _FULL((1, D)),
            pl.BlockSpec(memory_space=pl.ANY),
        ],
        out_specs=pl.BlockSpec((B, D), lambda i: (i + off, 0)),
        out_shape=jax.ShapeDtypeStruct(((T + 1) * B, D), jnp.float32),
        input_output_aliases={13: 0},
    )(frp, g01, g23, dts, ats,
      w0, w1, w2, w3, m2, b2, gm, bt, prev)


def _prep_params(rel_lin_w, rel_lin_b, rel_freq_w, rel_freq_b,
                 abs_lin_w, abs_lin_b, abs_freq_w, abs_freq_b,
                 time_proj_w, ctx_token, context_proj_w,
                 final_proj_w, final_proj_b, ln_gamma, ln_beta):
    wt = final_proj_w[:, 4 * D:5 * D]
    m = time_proj_w.T @ wt.T
    b2 = (final_proj_b + rel_lin_b[0] * m[0] + abs_lin_b[0] * m[8]).reshape(1, D)
    m2 = jnp.concatenate([
        (rel_lin_w[0, 0] * m[0]).reshape(1, D),
        (abs_lin_w[0, 0] * m[8]).reshape(1, D),
        m[1:8], m[9:16]], axis=0)
    frp = jnp.zeros((32,), jnp.float32)
    frp = frp.at[0:7].set(rel_freq_w[:, 0]).at[8:15].set(rel_freq_b)
    frp = frp.at[16:23].set(abs_freq_w[:, 0]).at[24:31].set(abs_freq_b)
    ws = [final_proj_w[:, k * D:(k + 1) * D].T.astype(jnp.bfloat16)
          for k in range(4)]
    cw = context_proj_w.T
    ct = ctx_token.reshape(1, D)
    gm = ln_gamma.reshape(1, D)
    bt = ln_beta.reshape(1, D)
    return frp, ws, m2, b2, cw, ct, gm, bt


def kernel(raw_concept_ids, concept_ids, value_ids, position_ids, delta_ts,
           abs_ts, patient_contexts, raw_table, con_table, val_table,
           pos_table, rel_lin_w, rel_lin_b, rel_freq_w, rel_freq_b,
           abs_lin_w, abs_lin_b, abs_freq_w, abs_freq_b, time_proj_w,
           ctx_token, context_proj_w, final_proj_w, final_proj_b,
           ln_gamma, ln_beta):
    id_in = (raw_concept_ids, concept_ids, value_ids, position_ids)
    starts = [sum(SIZES[:h]) for h in range(len(SIZES))]
    tabs = (raw_table, con_table, val_table, pos_table)
    gh, dts, ats = [], [], []
    for h, nt in enumerate(SIZES):
        s0 = starts[h]
        ids_h = [a[:, s0:s0 + nt].astype(jnp.int32).T.reshape(
            NW, nt * B // NW // CH, CH) for a in id_in]
        gh.append(_sc_gather(nt * B)(*tabs, *ids_h))
        dts.append(delta_ts[:, s0:s0 + nt].T.reshape(nt, 1, B))
        ats.append(abs_ts[:, s0:s0 + nt].T.reshape(nt, 1, B))
    frp, ws, m2, b2, cw, ct, gm, bt = _prep_params(
        rel_lin_w, rel_lin_b, rel_freq_w, rel_freq_b,
        abs_lin_w, abs_lin_b, abs_freq_w, abs_freq_b,
        time_proj_w, ctx_token, context_proj_w,
        final_proj_w, final_proj_b, ln_gamma, ln_beta)
    out = _tc_half1(frp, gh[0][0], gh[0][1], dts[0], ats[0], patient_contexts,
                    *ws, m2, b2, cw, ct, gm, bt, SIZES[0])
    for h in range(1, len(SIZES)):
        out = _tc_half2(frp, gh[h][0], gh[h][1], dts[h], ats[h],
                        *ws, m2, b2, gm, bt, out, 1 + starts[h], SIZES[h])
    return out.reshape(T + 1, B, D).transpose(1, 0, 2)
